# baseline (speedup 1.0000x reference)
import functools

import jax
import jax.numpy as jnp
from jax import lax
from jax.experimental import pallas as pl
from jax.experimental.pallas import tpu as pltpu
from jax.experimental.pallas import tpu_sc as plsc

_DN0 = (((0,), (0,)), ((), ()))
_DN01 = (((0,), (1,)), ((), ()))

_EDGE_BLOCK = 16000
_SC_WINDOW = 16000


def _prep_nodes_body(x_ref, w_in_ref, b_in_ref, w_top_ref, hv0t_ref, g0t_ref):
    hv0t = lax.dot_general(
        w_in_ref[...], x_ref[...], _DN01, preferred_element_type=jnp.float32
    ) + b_in_ref[...]
    hv0t_ref[...] = hv0t
    g0t_ref[...] = lax.dot_general(
        w_top_ref[...], hv0t, _DN0, preferred_element_type=jnp.float32
    )


def _prep_edges_body(ea_ref, w2_ref, b2_ref, eat0_ref, eat1_ref):
    m = lax.dot_general(
        w2_ref[...], ea_ref[...], _DN01, preferred_element_type=jnp.float32
    ) + b2_ref[...]
    half = m.shape[0] // 2
    eat0_ref[...] = m[:half]
    eat1_ref[...] = m[half:]


def _mid_body(pt_ref, hvt_ref, w_top_ref, hv1t_ref, g1t_ref):
    hv1t = jnp.maximum(pt_ref[...], 0.0) + hvt_ref[...]
    hv1t_ref[...] = hv1t
    g1t_ref[...] = lax.dot_general(
        w_top_ref[...], hv1t, _DN0, preferred_element_type=jnp.float32
    )


def _fin_body(pt_ref, hvt_ref, w_r_ref, b_r_ref, out_ref):
    hv2t = jnp.maximum(pt_ref[...], 0.0) + hvt_ref[...]
    out_ref[...] = jnp.maximum(
        lax.dot_general(hv2t, w_r_ref[...], _DN0, preferred_element_type=jnp.float32)
        + b_r_ref[...],
        0.0,
    )


def _sc_round(gt, eat, src, dst):
    n = gt.shape[1]
    e = src.shape[0]
    w = _SC_WINDOW
    mesh = plsc.VectorSubcoreMesh(core_axis_name="c", subcore_axis_name="s")

    @functools.partial(
        pl.kernel,
        out_type=jax.ShapeDtypeStruct((32, n), jnp.float32),
        mesh=mesh,
        compiler_params=pltpu.CompilerParams(needs_layout_passes=False),
        scratch_types=[
            pltpu.VMEM((n,), jnp.float32),
            pltpu.VMEM((n,), jnp.float32),
            pltpu.VMEM((w,), jnp.int32),
            pltpu.VMEM((w,), jnp.int32),
            pltpu.VMEM((w,), jnp.float32),
        ],
    )
    def k(gt_h, eat_h, src_h, dst_h, out_h, g_v, acc_v, src_v, dst_v, ea_v):
        c = lax.axis_index("c")
        s = lax.axis_index("s")
        f = s * 2 + c

        pltpu.sync_copy(gt_h.at[f], g_v)

        neg_inf = jnp.full((16,), -jnp.inf, dtype=jnp.float32)

        def init_body(i, carry):
            acc_v[pl.ds(i * 16, 16)] = neg_inf
            return carry

        lax.fori_loop(0, n // 16, init_body, 0)

        lane = lax.iota(jnp.int32, 16)

        def win_body(wi, carry):
            base = wi * w
            pltpu.sync_copy(src_h.at[pl.ds(base, w)], src_v)
            pltpu.sync_copy(dst_h.at[pl.ds(base, w)], dst_v)
            pltpu.sync_copy(eat_h.at[f, pl.ds(base, w)], ea_v)

            def vec_body(i, carry2):
                off = i * 16
                sv = src_v[pl.ds(off, 16)]
                dv = dst_v[pl.ds(off, 16)]
                ev = ea_v[pl.ds(off, 16)]
                gv = plsc.load_gather(g_v, [sv])
                z = gv + ev
                _, last = plsc.scan_count(dv)
                unique = jnp.all(last)

                def fast(_):
                    cur = plsc.load_gather(acc_v, [dv])
                    plsc.store_scatter(acc_v, [dv], jnp.maximum(cur, z))
                    return 0

                def slow(_):
                    for l in range(16):
                        m = lane == l
                        cur = plsc.load_gather(acc_v, [dv], mask=m)
                        plsc.store_scatter(
                            acc_v, [dv], jnp.maximum(cur, z), mask=m
                        )
                    return 0

                lax.cond(unique, fast, slow, 0)
                return carry2

            lax.fori_loop(0, w // 16, vec_body, 0)
            return carry

        lax.fori_loop(0, e // w, win_body, 0)
        pltpu.sync_copy(acc_v, out_h.at[f])

    return k(gt, eat, src, dst)


def kernel(x, edge_index, edge_attr, W_in, b_in, W_u0, b_u0, W_u1, b_u1, W_r, b_r):
    n, _ = x.shape
    e = edge_index.shape[1]
    out_dim = W_in.shape[1]

    src = edge_index[0].astype(jnp.int32)
    dst = edge_index[1].astype(jnp.int32)

    w2 = jnp.concatenate([W_u0[out_dim:], W_u1[out_dim:]], axis=1)
    b2 = jnp.concatenate([b_u0, b_u1]).reshape(-1, 1)

    hv0t, g0t = pl.pallas_call(
        _prep_nodes_body,
        out_shape=[jax.ShapeDtypeStruct((out_dim, n), jnp.float32)] * 2,
    )(x, W_in, b_in.reshape(-1, 1), W_u0[:out_dim])

    de = edge_attr.shape[1]
    blk = _EDGE_BLOCK
    eat0, eat1 = pl.pallas_call(
        _prep_edges_body,
        grid=(e // blk,),
        in_specs=[
            pl.BlockSpec((blk, de), lambda i: (i, 0)),
            pl.BlockSpec((de, 2 * out_dim), lambda i: (0, 0)),
            pl.BlockSpec((2 * out_dim, 1), lambda i: (0, 0)),
        ],
        out_specs=[
            pl.BlockSpec((out_dim, blk), lambda i: (0, i)),
            pl.BlockSpec((out_dim, blk), lambda i: (0, i)),
        ],
        out_shape=[jax.ShapeDtypeStruct((out_dim, e), jnp.float32)] * 2,
    )(edge_attr, w2, b2)

    p0 = _sc_round(g0t, eat0, src, dst)

    hv1t, g1t = pl.pallas_call(
        _mid_body,
        out_shape=[jax.ShapeDtypeStruct((out_dim, n), jnp.float32)] * 2,
    )(p0, hv0t, W_u1[:out_dim])

    p1 = _sc_round(g1t, eat1, src, dst)

    out = pl.pallas_call(
        _fin_body,
        out_shape=jax.ShapeDtypeStruct((n, out_dim), jnp.float32),
    )(p1, hv1t, W_r, b_r.reshape(1, -1))

    return out


# speedup vs baseline: 4.1808x; 4.1808x over previous
import functools

import jax
import jax.numpy as jnp
from jax import lax
from jax.experimental import pallas as pl
from jax.experimental.pallas import tpu as pltpu
from jax.experimental.pallas import tpu_sc as plsc

_DN0 = (((0,), (0,)), ((), ()))
_DN01 = (((0,), (1,)), ((), ()))

_EDGE_BLOCK = 16000
_SC_WINDOW = 3200


def _prep_nodes_body(x_ref, w_in_ref, b_in_ref, w_top_ref, hv0t_ref, g0t_ref):
    hv0t = lax.dot_general(
        w_in_ref[...], x_ref[...], _DN01, preferred_element_type=jnp.float32
    ) + b_in_ref[...]
    hv0t_ref[...] = hv0t
    g0t_ref[...] = lax.dot_general(
        w_top_ref[...], hv0t, _DN0, preferred_element_type=jnp.float32
    )


def _prep_edges_body(ea_ref, w2_ref, b2_ref, eat0_ref, eat1_ref):
    m = lax.dot_general(
        w2_ref[...], ea_ref[...], _DN01, preferred_element_type=jnp.float32
    ) + b2_ref[...]
    half = m.shape[0] // 2
    eat0_ref[...] = m[:half]
    eat1_ref[...] = m[half:]


def _mid_body(pt_ref, hvt_ref, w_top_ref, hv1t_ref, g1t_ref):
    pooled = jnp.max(pt_ref[...], axis=0)
    hv1t = jnp.maximum(pooled, 0.0) + hvt_ref[...]
    hv1t_ref[...] = hv1t
    g1t_ref[...] = lax.dot_general(
        w_top_ref[...], hv1t, _DN0, preferred_element_type=jnp.float32
    )


def _fin_body(pt_ref, hvt_ref, w_r_ref, b_r_ref, out_ref):
    pooled = jnp.max(pt_ref[...], axis=0)
    hv2t = jnp.maximum(pooled, 0.0) + hvt_ref[...]
    out_ref[...] = jnp.maximum(
        lax.dot_general(hv2t, w_r_ref[...], _DN0, preferred_element_type=jnp.float32)
        + b_r_ref[...],
        0.0,
    )


_K = 4
_C = 4


def _sc_round(gt, eat, src, dst):
    n = gt.shape[1]
    e = src.shape[0]
    w = _SC_WINDOW
    ec = e // _C
    mesh = plsc.VectorSubcoreMesh(core_axis_name="c", subcore_axis_name="s")

    @functools.partial(
        pl.kernel,
        out_type=jax.ShapeDtypeStruct((_C * 32, n), jnp.float32),
        mesh=mesh,
        compiler_params=pltpu.CompilerParams(needs_layout_passes=False),
        scratch_types=[pltpu.VMEM((n,), jnp.float32)] * 8
        + [pltpu.VMEM((w,), jnp.int32)] * 2
        + [pltpu.VMEM((w,), jnp.float32)] * 8,
    )
    def k(gt_h, eat_h, src_h, dst_h, out_h, *scr):
        gs = scr[0:4]
        accs = scr[4:8]
        src_v, dst_v = scr[8:10]
        eas = scr[10:14]
        zs = scr[14:18]
        c = lax.axis_index("c")
        s = lax.axis_index("s")
        wid = s * 2 + c
        grp = wid // _C
        j = wid % _C
        f0 = grp * _K

        for ki in range(_K):
            pltpu.sync_copy(gt_h.at[f0 + ki], gs[ki])

        neg_inf = jnp.full((16,), -jnp.inf, dtype=jnp.float32)

        @plsc.parallel_loop(0, n // 16, unroll=4)
        def _init(i):
            for ki in range(_K):
                accs[ki][pl.ds(i * 16, 16)] = neg_inf

        lane = lax.iota(jnp.int32, 16)
        base0 = j * ec

        def win_body(wi, carry):
            base = pl.multiple_of(base0 + wi * w, 128)
            pltpu.sync_copy(src_h.at[pl.ds(base, w)], src_v)
            pltpu.sync_copy(dst_h.at[pl.ds(base, w)], dst_v)
            for ki in range(_K):
                pltpu.sync_copy(eat_h.at[f0 + ki, pl.ds(base, w)], eas[ki])

            @plsc.parallel_loop(0, w // 16, unroll=2)
            def _zloop(i):
                off = i * 16
                sv = src_v[pl.ds(off, 16)]
                for ki in range(_K):
                    zs[ki][pl.ds(off, 16)] = (
                        plsc.load_gather(gs[ki], [sv]) + eas[ki][pl.ds(off, 16)]
                    )

            def vec_body(i, carry2):
                off = i * 16
                dv = dst_v[pl.ds(off, 16)]
                zv = [zs[ki][pl.ds(off, 16)] for ki in range(_K)]
                _, last = plsc.scan_count(dv)
                unique = jnp.all(last)

                def fast(_):
                    for ki in range(_K):
                        cur = plsc.load_gather(accs[ki], [dv])
                        plsc.store_scatter(accs[ki], [dv], jnp.maximum(cur, zv[ki]))
                    return 0

                def slow(_):
                    for l in range(16):
                        m = lane == l
                        for ki in range(_K):
                            cur = plsc.load_gather(accs[ki], [dv], mask=m)
                            plsc.store_scatter(
                                accs[ki], [dv], jnp.maximum(cur, zv[ki]), mask=m
                            )
                    return 0

                lax.cond(unique, fast, slow, 0)
                return carry2

            lax.fori_loop(0, w // 16, vec_body, 0)
            return carry

        lax.fori_loop(0, ec // w, win_body, 0)
        for ki in range(_K):
            pltpu.sync_copy(accs[ki], out_h.at[j * 32 + f0 + ki])

    return k(gt, eat, src, dst).reshape(_C, 32, n)


def kernel(x, edge_index, edge_attr, W_in, b_in, W_u0, b_u0, W_u1, b_u1, W_r, b_r):
    n, _ = x.shape
    e = edge_index.shape[1]
    out_dim = W_in.shape[1]

    src = edge_index[0].astype(jnp.int32)
    dst = edge_index[1].astype(jnp.int32)

    w2 = jnp.concatenate([W_u0[out_dim:], W_u1[out_dim:]], axis=1)
    b2 = jnp.concatenate([b_u0, b_u1]).reshape(-1, 1)

    hv0t, g0t = pl.pallas_call(
        _prep_nodes_body,
        out_shape=[jax.ShapeDtypeStruct((out_dim, n), jnp.float32)] * 2,
    )(x, W_in, b_in.reshape(-1, 1), W_u0[:out_dim])

    de = edge_attr.shape[1]
    blk = _EDGE_BLOCK
    eat0, eat1 = pl.pallas_call(
        _prep_edges_body,
        grid=(e // blk,),
        in_specs=[
            pl.BlockSpec((blk, de), lambda i: (i, 0)),
            pl.BlockSpec((de, 2 * out_dim), lambda i: (0, 0)),
            pl.BlockSpec((2 * out_dim, 1), lambda i: (0, 0)),
        ],
        out_specs=[
            pl.BlockSpec((out_dim, blk), lambda i: (0, i)),
            pl.BlockSpec((out_dim, blk), lambda i: (0, i)),
        ],
        out_shape=[jax.ShapeDtypeStruct((out_dim, e), jnp.float32)] * 2,
    )(edge_attr, w2, b2)

    p0 = _sc_round(g0t, eat0, src, dst)

    hv1t, g1t = pl.pallas_call(
        _mid_body,
        out_shape=[jax.ShapeDtypeStruct((out_dim, n), jnp.float32)] * 2,
    )(p0, hv0t, W_u1[:out_dim])

    p1 = _sc_round(g1t, eat1, src, dst)

    out = pl.pallas_call(
        _fin_body,
        out_shape=jax.ShapeDtypeStruct((n, out_dim), jnp.float32),
    )(p1, hv1t, W_r, b_r.reshape(1, -1))

    return out
